# baseline (speedup 1.0000x reference)
import functools

import jax
import jax.numpy as jnp
from jax import lax
from jax.experimental import pallas as pl
from jax.experimental.pallas import tpu as pltpu
from jax.experimental.pallas import tpu_sc as plsc

N_ROWS = 524288
N_COLS = 256
IGNORE = -100

NUM_WORKERS = 32
ROWS_PER_WORKER = N_ROWS // NUM_WORKERS
CHUNK = 128
NCHUNKS = ROWS_PER_WORKER // CHUNK
VPC = CHUNK // 16


def _sc_partials(flat_logits, target):
    mesh = plsc.VectorSubcoreMesh(core_axis_name="c", subcore_axis_name="s")

    @functools.partial(
        pl.kernel,
        out_type=jax.ShapeDtypeStruct((NUM_WORKERS, 32), jnp.float32),
        mesh=mesh,
        scratch_types=[
            pltpu.VMEM((ROWS_PER_WORKER,), jnp.int32),
            pltpu.VMEM((CHUNK,), jnp.int32),
            pltpu.VMEM((CHUNK,), jnp.int32),
            pltpu.VMEM((CHUNK,), jnp.float32),
            pltpu.VMEM((CHUNK,), jnp.float32),
            pltpu.VMEM((32,), jnp.float32),
            pltpu.SemaphoreType.DMA,
            pltpu.SemaphoreType.DMA,
        ],
    )
    def k(logits_hbm, tgt_hbm, out_hbm, tgt_v, idx0, idx1, val0, val1,
          part_v, sem0, sem1):
        wid = lax.axis_index("s") * 2 + lax.axis_index("c")
        base = wid * ROWS_PER_WORKER
        pltpu.sync_copy(tgt_hbm.at[pl.ds(base, ROWS_PER_WORKER)], tgt_v)

        lanes = lax.iota(jnp.int32, 16)

        def fill_idx(j, idx_v):
            for u in range(VPC):
                t16 = tgt_v[pl.ds(j * CHUNK + u * 16, 16)]
                safe = jnp.where(t16 != IGNORE, t16, 0)
                row = (base + j * CHUNK + u * 16) + lanes
                idx_v[pl.ds(u * 16, 16)] = row * N_COLS + safe

        def start(idx_v, val_v, sem):
            pltpu.async_copy(logits_hbm.at[idx_v], val_v, sem)

        def wait(idx_v, val_v, sem):
            pltpu.make_async_copy(logits_hbm.at[idx_v], val_v, sem).wait()

        def accum(j, val_v, acc, cnt):
            for u in range(VPC):
                x = val_v[pl.ds(u * 16, 16)]
                t16 = tgt_v[pl.ds(j * CHUNK + u * 16, 16)]
                m = t16 != IGNORE
                e = 1.0 - x
                acc = acc + jnp.where(m, e * e, 0.0)
                cnt = cnt + jnp.where(m, 1.0, 0.0)
            return acc, cnt

        fill_idx(0, idx0)
        start(idx0, val0, sem0)

        def body(j2, carry):
            acc, cnt = carry
            a = 2 * j2
            b = a + 1
            fill_idx(b, idx1)
            start(idx1, val1, sem1)
            wait(idx0, val0, sem0)
            acc, cnt = accum(a, val0, acc, cnt)

            @pl.when(b + 1 < NCHUNKS)
            def _():
                fill_idx(b + 1, idx0)
                start(idx0, val0, sem0)

            wait(idx1, val1, sem1)
            acc, cnt = accum(b, val1, acc, cnt)
            return acc, cnt

        zero = jnp.zeros((16,), jnp.float32)
        acc, cnt = lax.fori_loop(0, NCHUNKS // 2, body, (zero, zero))
        part_v[pl.ds(0, 16)] = acc
        part_v[pl.ds(16, 16)] = cnt
        pltpu.sync_copy(part_v, out_hbm.at[wid])

    return k(flat_logits, target)


def _tc_finish(partials):
    def body(p_ref, o_ref):
        p = p_ref[...]
        s = jnp.sum(p[:, :16])
        c = jnp.sum(p[:, 16:])
        o_ref[0, 0] = s / c

    out = pl.pallas_call(
        body,
        out_shape=jax.ShapeDtypeStruct((1, 1), jnp.float32),
        out_specs=pl.BlockSpec(memory_space=pltpu.SMEM),
    )(partials)
    return out[0, 0]


def kernel(contrast_logits, contrast_target):
    flat = contrast_logits.reshape(-1)
    tgt = contrast_target.astype(jnp.int32)
    partials = _sc_partials(flat, tgt)
    return _tc_finish(partials)


# speedup vs baseline: 1.0670x; 1.0670x over previous
import functools

import jax
import jax.numpy as jnp
from jax import lax
from jax.experimental import pallas as pl
from jax.experimental.pallas import tpu as pltpu
from jax.experimental.pallas import tpu_sc as plsc

N_ROWS = 524288
N_COLS = 256
IGNORE = -100

NUM_WORKERS = 32
ROWS_PER_WORKER = N_ROWS // NUM_WORKERS
CHUNK = 128
NCHUNKS = ROWS_PER_WORKER // CHUNK
VPC = CHUNK // 16


NBUF = 16
NGROUPS = NCHUNKS // NBUF


def _sc_partials(flat_logits, target):
    mesh = plsc.VectorSubcoreMesh(core_axis_name="c", subcore_axis_name="s")

    scratch = (
        [pltpu.VMEM((ROWS_PER_WORKER,), jnp.int32)]
        + [pltpu.VMEM((CHUNK,), jnp.int32) for _ in range(NBUF)]
        + [pltpu.VMEM((CHUNK,), jnp.float32) for _ in range(NBUF)]
        + [pltpu.VMEM((32,), jnp.float32)]
        + [pltpu.SemaphoreType.DMA for _ in range(NBUF)]
    )

    @functools.partial(
        pl.kernel,
        out_type=jax.ShapeDtypeStruct((NUM_WORKERS, 32), jnp.float32),
        mesh=mesh,
        scratch_types=scratch,
    )
    def k(logits_hbm, tgt_hbm, out_hbm, *refs):
        tgt_v = refs[0]
        idx = refs[1:1 + NBUF]
        val = refs[1 + NBUF:1 + 2 * NBUF]
        part_v = refs[1 + 2 * NBUF]
        sem = refs[2 + 2 * NBUF:2 + 3 * NBUF]

        wid = lax.axis_index("s") * 2 + lax.axis_index("c")
        base = wid * ROWS_PER_WORKER
        pltpu.sync_copy(tgt_hbm.at[pl.ds(base, ROWS_PER_WORKER)], tgt_v)

        lanes = lax.iota(jnp.int32, 16)

        def fill_and_start(j, b):
            for u in range(VPC):
                t16 = tgt_v[pl.ds(j * CHUNK + u * 16, 16)]
                safe = jnp.where(t16 != IGNORE, t16, 0)
                row = (base + j * CHUNK + u * 16) + lanes
                idx[b][pl.ds(u * 16, 16)] = row * N_COLS + safe
            pltpu.async_copy(logits_hbm.at[idx[b]], val[b], sem[b])

        def wait_accum(j, b, acc, cnt):
            pltpu.make_async_copy(
                logits_hbm.at[idx[b]], val[b], sem[b]).wait()
            for u in range(VPC):
                x = val[b][pl.ds(u * 16, 16)]
                t16 = tgt_v[pl.ds(j * CHUNK + u * 16, 16)]
                m = t16 != IGNORE
                e = 1.0 - x
                acc = acc + jnp.where(m, e * e, 0.0)
                cnt = cnt + jnp.where(m, 1.0, 0.0)
            return acc, cnt

        for b in range(NBUF):
            fill_and_start(b, b)

        def body(g, carry):
            acc, cnt = carry
            for b in range(NBUF):
                j = g * NBUF + b
                acc, cnt = wait_accum(j, b, acc, cnt)
                fill_and_start(j + NBUF, b)
            return acc, cnt

        zero = jnp.zeros((16,), jnp.float32)
        acc, cnt = lax.fori_loop(0, NGROUPS - 1, body, (zero, zero))
        for b in range(NBUF):
            j = (NGROUPS - 1) * NBUF + b
            acc, cnt = wait_accum(j, b, acc, cnt)

        part_v[pl.ds(0, 16)] = acc
        part_v[pl.ds(16, 16)] = cnt
        pltpu.sync_copy(part_v, out_hbm.at[wid])

    return k(flat_logits, target)


def _tc_finish(partials):
    def body(p_ref, o_ref):
        p = p_ref[...]
        s = jnp.sum(p[:, :16])
        c = jnp.sum(p[:, 16:])
        o_ref[0, 0] = s / c

    out = pl.pallas_call(
        body,
        out_shape=jax.ShapeDtypeStruct((1, 1), jnp.float32),
        out_specs=pl.BlockSpec(memory_space=pltpu.SMEM),
    )(partials)
    return out[0, 0]


def kernel(contrast_logits, contrast_target):
    flat = contrast_logits.reshape(-1)
    tgt = contrast_target.astype(jnp.int32)
    partials = _sc_partials(flat, tgt)
    return _tc_finish(partials)
